# baseline (speedup 1.0000x reference)
import jax
import jax.numpy as jnp
from jax.experimental import pallas as pl
from jax.experimental.pallas import tpu as pltpu

B, S = 4, 2048
IN_F, OUT_F = 4096, 4096
K_OUT = 204
BO = 256
RB = 8
NB = BO // RB
KFLAT = RB * K_OUT
HI = 32
LO = 128
BM = 1024
BN = 1024


def _build_w_kernel(packed_ref, scales_ref, vals_ref, idx_ref, w_ref, acc_ref):
    p = packed_ref[...]
    s = scales_ref[...]
    lo_q = (p & 0xF) - 8
    hi_q = ((p >> 4) & 0xF) - 8
    acc_ref[:, 0:IN_F // 2] = lo_q.astype(jnp.float32) * s
    acc_ref[:, IN_F // 2:IN_F] = hi_q.astype(jnp.float32) * s

    r_pat = jax.lax.broadcasted_iota(jnp.int32, (1, KFLAT), 1) // K_OUT
    iota_code = jax.lax.broadcasted_iota(
        jnp.int32, (HI * RB, KFLAT), 0).astype(jnp.bfloat16)
    iota_lo = jax.lax.broadcasted_iota(
        jnp.int32, (LO, KFLAT), 0).astype(jnp.bfloat16)
    zero_bf = jnp.zeros((), jnp.bfloat16)
    for b in range(NB):
        idx = idx_ref[b:b + 1, :]
        vals = vals_ref[b:b + 1, :].astype(jnp.bfloat16)
        idx_p = (idx >> 1) + (idx & 1) * (IN_F // 2)
        hi = idx_p >> 7
        lo = (idx_p & (LO - 1)).astype(jnp.bfloat16)
        code = (hi * RB + r_pat).astype(jnp.bfloat16)
        a_t = jnp.where(iota_code == code, vals, zero_bf)
        b_t = jnp.where(iota_lo == lo, jnp.bfloat16(1.0), zero_bf)
        c = jax.lax.dot_general(
            a_t, b_t, (((1,), (1,)), ((), ())),
            preferred_element_type=jnp.float32)
        for h in range(HI):
            r0, r1 = RB * b, RB * (b + 1)
            c0, c1 = LO * h, LO * (h + 1)
            acc_ref[r0:r1, c0:c1] = (
                c[RB * h:RB * (h + 1), :] + acc_ref[r0:r1, c0:c1])

    iota128 = jax.lax.broadcasted_iota(jnp.int32, (BO, 128), 1)
    half_idx = iota128 >> 1
    par = iota128 & 1
    for c in range(IN_F // 128):
        base = 128 * (c // 2)
        gidx = 64 * (c % 2) + half_idx
        lo_g = jnp.take_along_axis(acc_ref[:, base:base + 128], gidx, axis=1)
        hi_g = jnp.take_along_axis(
            acc_ref[:, IN_F // 2 + base:IN_F // 2 + base + 128], gidx, axis=1)
        w_ref[:, 128 * c:128 * (c + 1)] = jnp.where(
            par == 0, lo_g, hi_g).astype(jnp.bfloat16)


def _matmul_kernel(x_ref, w_ref, o_ref):
    o_ref[...] = jax.lax.dot_general(
        x_ref[...], w_ref[...], (((1,), (1,)), ((), ())),
        preferred_element_type=jnp.float32)


def kernel(x, base_packed, scales, ortho_vals, ortho_idx):
    xf = x.reshape(B * S, IN_F)
    x_bf = xf.astype(jnp.bfloat16)
    scales2d = scales[:, None]
    vals_flat = ortho_vals.astype(jnp.float32).reshape(OUT_F // RB, KFLAT)
    idx_flat = ortho_idx.reshape(OUT_F // RB, KFLAT)

    w = pl.pallas_call(
        _build_w_kernel,
        out_shape=jax.ShapeDtypeStruct((OUT_F, IN_F), jnp.bfloat16),
        grid=(OUT_F // BO,),
        in_specs=[
            pl.BlockSpec((BO, IN_F // 2), lambda i: (i, 0)),
            pl.BlockSpec((BO, 1), lambda i: (i, 0)),
            pl.BlockSpec((BO // RB, KFLAT), lambda i: (i, 0)),
            pl.BlockSpec((BO // RB, KFLAT), lambda i: (i, 0)),
        ],
        out_specs=pl.BlockSpec((BO, IN_F), lambda i: (i, 0)),
        scratch_shapes=[pltpu.VMEM((BO, IN_F), jnp.float32)],
        compiler_params=pltpu.CompilerParams(
            dimension_semantics=("parallel",),
            vmem_limit_bytes=48 * 1024 * 1024,
        ),
        name="build_w",
    )(base_packed, scales2d, vals_flat, idx_flat)

    out = pl.pallas_call(
        _matmul_kernel,
        out_shape=jax.ShapeDtypeStruct((B * S, OUT_F), jnp.float32),
        grid=(OUT_F // BN, B * S // BM),
        in_specs=[
            pl.BlockSpec((BM, IN_F), lambda i, j: (j, 0)),
            pl.BlockSpec((BN, IN_F), lambda i, j: (i, 0)),
        ],
        out_specs=pl.BlockSpec((BM, BN), lambda i, j: (j, i)),
        compiler_params=pltpu.CompilerParams(
            dimension_semantics=("parallel", "arbitrary"),
            vmem_limit_bytes=48 * 1024 * 1024,
        ),
        name="int4_ortho_matmul",
    )(x_bf, w)

    return out.reshape(B, S, OUT_F)


# speedup vs baseline: 1.1120x; 1.1120x over previous
import jax
import jax.numpy as jnp
from jax.experimental import pallas as pl
from jax.experimental.pallas import tpu as pltpu

B, S = 4, 2048
IN_F, OUT_F = 4096, 4096
K_OUT = 204
BO = 256
RB = 8
NB = BO // RB
KFLAT = RB * K_OUT
HI = 32
LO = 128
BM = 1024
BN = 1024


def _build_w_kernel(packed_ref, scales_ref, vals_ref, idx_ref, x_ref,
                    w_ref, xbf_ref, acc_ref):
    xbf_ref[...] = x_ref[...].astype(jnp.bfloat16)
    p = packed_ref[...]
    s = scales_ref[...]
    lo_q = (p & 0xF) - 8
    hi_q = ((p >> 4) & 0xF) - 8
    acc_ref[:, 0:IN_F // 2] = lo_q.astype(jnp.float32) * s
    acc_ref[:, IN_F // 2:IN_F] = hi_q.astype(jnp.float32) * s

    r_pat = jax.lax.broadcasted_iota(jnp.int32, (1, KFLAT), 1) // K_OUT
    iota_code = jax.lax.broadcasted_iota(
        jnp.int32, (HI * RB, KFLAT), 0).astype(jnp.bfloat16)
    iota_lo = jax.lax.broadcasted_iota(
        jnp.int32, (LO, KFLAT), 0).astype(jnp.bfloat16)
    zero_bf = jnp.zeros((), jnp.bfloat16)
    for b in range(NB):
        idx = idx_ref[b:b + 1, :]
        vals = vals_ref[b:b + 1, :].astype(jnp.bfloat16)
        idx_p = (idx >> 1) + (idx & 1) * (IN_F // 2)
        hi = idx_p >> 7
        lo = (idx_p & (LO - 1)).astype(jnp.bfloat16)
        code = (hi * RB + r_pat).astype(jnp.bfloat16)
        a_t = jnp.where(iota_code == code, vals, zero_bf)
        b_t = jnp.where(iota_lo == lo, jnp.bfloat16(1.0), zero_bf)
        c = jax.lax.dot_general(
            a_t, b_t, (((1,), (1,)), ((), ())),
            preferred_element_type=jnp.float32)
        for h in range(HI):
            r0, r1 = RB * b, RB * (b + 1)
            c0, c1 = LO * h, LO * (h + 1)
            acc_ref[r0:r1, c0:c1] = (
                c[RB * h:RB * (h + 1), :] + acc_ref[r0:r1, c0:c1])

    iota128 = jax.lax.broadcasted_iota(jnp.int32, (BO, 128), 1)
    half_idx = iota128 >> 1
    par = iota128 & 1
    for c in range(IN_F // 128):
        base = 128 * (c // 2)
        gidx = 64 * (c % 2) + half_idx
        lo_g = jnp.take_along_axis(acc_ref[:, base:base + 128], gidx, axis=1)
        hi_g = jnp.take_along_axis(
            acc_ref[:, IN_F // 2 + base:IN_F // 2 + base + 128], gidx, axis=1)
        w_ref[:, 128 * c:128 * (c + 1)] = jnp.where(
            par == 0, lo_g, hi_g).astype(jnp.bfloat16)


def _matmul_kernel(x_ref, w_ref, o_ref):
    o_ref[...] = jax.lax.dot_general(
        x_ref[...], w_ref[...], (((1,), (1,)), ((), ())),
        preferred_element_type=jnp.float32)


def kernel(x, base_packed, scales, ortho_vals, ortho_idx):
    xf = x.reshape(B * S, IN_F)
    scales2d = scales[:, None]
    vals_flat = ortho_vals.astype(jnp.float32).reshape(OUT_F // RB, KFLAT)
    idx_flat = ortho_idx.reshape(OUT_F // RB, KFLAT)

    n_blk = OUT_F // BO
    xrows = B * S // n_blk
    w, x_bf = pl.pallas_call(
        _build_w_kernel,
        out_shape=[
            jax.ShapeDtypeStruct((OUT_F, IN_F), jnp.bfloat16),
            jax.ShapeDtypeStruct((B * S, IN_F), jnp.bfloat16),
        ],
        grid=(n_blk,),
        in_specs=[
            pl.BlockSpec((BO, IN_F // 2), lambda i: (i, 0)),
            pl.BlockSpec((BO, 1), lambda i: (i, 0)),
            pl.BlockSpec((BO // RB, KFLAT), lambda i: (i, 0)),
            pl.BlockSpec((BO // RB, KFLAT), lambda i: (i, 0)),
            pl.BlockSpec((xrows, IN_F), lambda i: (i, 0)),
        ],
        out_specs=[
            pl.BlockSpec((BO, IN_F), lambda i: (i, 0)),
            pl.BlockSpec((xrows, IN_F), lambda i: (i, 0)),
        ],
        scratch_shapes=[pltpu.VMEM((BO, IN_F), jnp.float32)],
        compiler_params=pltpu.CompilerParams(
            dimension_semantics=("parallel",),
            vmem_limit_bytes=48 * 1024 * 1024,
        ),
        name="build_w",
    )(base_packed, scales2d, vals_flat, idx_flat, xf)

    out = pl.pallas_call(
        _matmul_kernel,
        out_shape=jax.ShapeDtypeStruct((B * S, OUT_F), jnp.float32),
        grid=(OUT_F // BN, B * S // BM),
        in_specs=[
            pl.BlockSpec((BM, IN_F), lambda i, j: (j, 0)),
            pl.BlockSpec((BN, IN_F), lambda i, j: (i, 0)),
        ],
        out_specs=pl.BlockSpec((BM, BN), lambda i, j: (j, i)),
        compiler_params=pltpu.CompilerParams(
            dimension_semantics=("parallel", "arbitrary"),
            vmem_limit_bytes=48 * 1024 * 1024,
        ),
        name="int4_ortho_matmul",
    )(x_bf, w)

    return out.reshape(B, S, OUT_F)
